# baseline (speedup 1.0000x reference)
import functools

import jax
import jax.numpy as jnp
from jax import lax
from jax.experimental import pallas as pl
from jax.experimental.pallas import tpu as pltpu
from jax.experimental.pallas import tpu_sc as plsc

NSTATES = 32
HID = 64
NTOT = 1048576
NC = 2
NS = 16
L = 16
NW = NC * NS
PER_W = NTOT // NW
NVEC = PER_W // L
ROWS = NSTATES + 2
UNROLL = 8

NCH = 8
CHN = PER_W // NCH
CVEC = CHN // L


@functools.cache
def _build_sc_partials():
    mesh = plsc.VectorSubcoreMesh(
        core_axis_name="c", subcore_axis_name="s",
        num_cores=NC, num_subcores=NS)

    @functools.partial(
        pl.kernel,
        out_type=jax.ShapeDtypeStruct((ROWS, NW, L), jnp.float32),
        mesh=mesh,
        scratch_types=[
            pltpu.VMEM((PER_W,), jnp.int32),
            pltpu.VMEM((PER_W,), jnp.float32),
            pltpu.VMEM((ROWS, L), jnp.float32),
            pltpu.SemaphoreType.DMA,
            pltpu.SemaphoreType.DMA,
        ],
        compiler_params=pltpu.CompilerParams(needs_layout_passes=False),
    )
    def _sc_partials(states_hbm, phases_hbm, out_hbm, st_v, ph_v, blk_v,
                     sem0, sem1):
        c = lax.axis_index("c")
        s = lax.axis_index("s")
        w = s * NC + c
        base = w * PER_W
        d_st = pltpu.async_copy(
            states_hbm.at[pl.ds(base, PER_W)], st_v, sem0)
        d_ph = pltpu.async_copy(
            phases_hbm.at[pl.ds(base, PER_W)], ph_v, sem1)

        zero = jnp.zeros((L,), jnp.float32)
        for r in range(ROWS):
            blk_v[r] = zero
        lanes = lax.broadcasted_iota(jnp.int32, (L,), 0)
        ones = jnp.ones((L,), jnp.float32)

        d_st.wait()

        def hist_body(i):
            sv = st_v[pl.ds(i * L, L)]
            plsc.addupdate_scatter(blk_v, [sv, lanes], ones)

        plsc.parallel_loop(0, NVEC, 1, unroll=UNROLL)(hist_body)

        d_ph.wait()

        def mom_body(i, cr):
            a1, a2 = cr
            pv = ph_v[pl.ds(i * L, L)]
            return (a1 + pv, a2 + pv * pv)

        a1, a2 = plsc.parallel_loop(
            0, NVEC, 1, unroll=UNROLL, carry=(zero, zero))(mom_body)
        blk_v[NSTATES] = a1
        blk_v[NSTATES + 1] = a2

        pltpu.sync_copy(blk_v, out_hbm.at[:, w])

    return _sc_partials


def _poly(t, coeffs):
    acc = jnp.float32(coeffs[-1])
    for c in coeffs[-2::-1]:
        acc = acc * t + jnp.float32(c)
    return acc


_I0_SMALL = (1.0, 3.5156229, 3.0899424, 1.2067492, 0.2659732,
             0.0360768, 0.0045813)
_I0_LARGE = (0.39894228, 0.01328592, 0.00225319, -0.00157565, 0.00916281,
             -0.02057706, 0.02635537, -0.01647633, 0.00392377)
_I1_SMALL = (0.5, 0.87890594, 0.51498869, 0.15084934, 0.02658733,
             0.00301532, 0.00032411)
_I1_LARGE = (0.39894228, -0.03988024, -0.00362018, 0.00163801, -0.01031555,
             0.02282967, -0.02895312, 0.01787654, -0.00420059)


def _i0e(x):
    xs = jnp.minimum(x, 3.75)
    xl = jnp.maximum(x, 3.75)
    small = _poly((xs / 3.75) ** 2, _I0_SMALL) * jnp.exp(-xs)
    large = _poly(3.75 / xl, _I0_LARGE) / jnp.sqrt(xl)
    return jnp.where(x < 3.75, small, large)


def _i1e(x):
    xs = jnp.minimum(x, 3.75)
    xl = jnp.maximum(x, 3.75)
    small = xs * _poly((xs / 3.75) ** 2, _I1_SMALL) * jnp.exp(-xs)
    large = _poly(3.75 / xl, _I1_LARGE) / jnp.sqrt(xl)
    return jnp.where(x < 3.75, small, large)


def _log_i0e(x):
    return jnp.log(_i0e(x))


LOG_2PI = 1.8378770664093453
N_POW = float(NTOT) ** (-0.2)


def _tc_tail(red_ref, emb_ref, lng_ref, lnb_ref, w1_ref, b1_ref,
             wk_ref, bk_ref, o_mi, o_hz, o_hphi, o_hcond, o_bdc):
    red = red_ref[...]
    sums = jnp.sum(red, axis=1)
    counts = sums[:NSTATES]
    s1 = sums[NSTATES]
    s2 = sums[NSTATES + 1]
    n = jnp.float32(NTOT)

    probs = counts / n + 1e-10
    h_z = -jnp.sum(probs * jnp.log(probs))

    var_p = (s2 - s1 * s1 / n) / (n - 1.0)
    std_p = jnp.sqrt(jnp.maximum(var_p, 0.0))
    bw = 1.06 * std_p * N_POW
    kap_kde = jnp.minimum(1.0 / (bw * bw + 1e-6), 100.0)
    h_phi = LOG_2PI + _log_i0e(kap_kde) + kap_kde

    h = emb_ref[...]
    mean = jnp.mean(h, axis=1, keepdims=True)
    var = jnp.mean((h - mean) ** 2, axis=1, keepdims=True)
    h = (h - mean) / jnp.sqrt(var + 1e-5) * lng_ref[...] + lnb_ref[...]
    h = jnp.maximum(h, 0.0)
    h = lax.dot_general(h, w1_ref[...], (((1,), (1,)), ((), ())),
                        preferred_element_type=jnp.float32) + b1_ref[...]
    h = jnp.maximum(h, 0.0)
    kp = lax.dot_general(h, wk_ref[...], (((1,), (1,)), ((), ())),
                         preferred_element_type=jnp.float32)[:, 0] + bk_ref[...]
    kappa = jnp.maximum(kp, 0.0) + jnp.log1p(jnp.exp(-jnp.abs(kp))) + 0.1

    i0e_k = _i0e(kappa)
    ratio = _i1e(kappa) / i0e_k
    h_vm = LOG_2PI + jnp.log(i0e_k) + kappa - kappa * ratio
    h_cond = jnp.sum((counts / n) * h_vm)

    mi = h_phi - h_cond
    bdc = jnp.clip(2.0 * mi / (h_z + h_phi + 1e-12), 0.0, 1.0)

    o_mi[...] = mi
    o_hz[...] = h_z
    o_hphi[...] = h_phi
    o_hcond[...] = h_cond
    o_bdc[...] = bdc


def kernel(states, phases, emb, ln_g, ln_b, W1, b1, Wmu, bmu, Wk, bk):
    partials = _build_sc_partials()(states, phases)
    red = partials.reshape(ROWS, NW * L)
    scalar = jax.ShapeDtypeStruct((), jnp.float32)
    out = pl.pallas_call(
        _tc_tail,
        out_shape=[scalar] * 5,
        out_specs=[pl.BlockSpec(memory_space=pltpu.SMEM)] * 5,
    )(red, emb, ln_g, ln_b, W1, b1, Wk, bk)
    return tuple(out)


# speedup vs baseline: 1.0614x; 1.0614x over previous
import functools

import jax
import jax.numpy as jnp
from jax import lax
from jax.experimental import pallas as pl
from jax.experimental.pallas import tpu as pltpu
from jax.experimental.pallas import tpu_sc as plsc

NSTATES = 32
HID = 64
NTOT = 1048576
NC = 2
NS = 16
L = 16
NW = NC * NS
PER_W = NTOT // NW
NVEC = PER_W // L
ROWS = NSTATES + 2
UNROLL = 8

NCH = 8
CHN = PER_W // NCH
CVEC = CHN // L


@functools.cache
def _build_sc_partials():
    mesh = plsc.VectorSubcoreMesh(
        core_axis_name="c", subcore_axis_name="s",
        num_cores=NC, num_subcores=NS)

    @functools.partial(
        pl.kernel,
        out_type=jax.ShapeDtypeStruct((ROWS, NW, L), jnp.float32),
        mesh=mesh,
        scratch_types=[
            pltpu.VMEM((PER_W,), jnp.int32),
            pltpu.VMEM((PER_W,), jnp.float32),
            pltpu.VMEM((ROWS, L), jnp.float32),
            pltpu.SemaphoreType.DMA,
            pltpu.SemaphoreType.DMA,
        ],
        compiler_params=pltpu.CompilerParams(needs_layout_passes=False),
    )
    def _sc_partials(states_hbm, phases_hbm, out_hbm, st_v, ph_v, blk_v,
                     sem0, sem1):
        c = lax.axis_index("c")
        s = lax.axis_index("s")
        w = s * NC + c
        base = w * PER_W
        d_st = pltpu.async_copy(
            states_hbm.at[pl.ds(base, PER_W)], st_v, sem0)
        d_ph = pltpu.async_copy(
            phases_hbm.at[pl.ds(base, PER_W)], ph_v, sem1)

        zero = jnp.zeros((L,), jnp.float32)
        for r in range(ROWS):
            blk_v[r] = zero
        lanes = lax.broadcasted_iota(jnp.int32, (L,), 0)
        ones = jnp.ones((L,), jnp.float32)

        d_st.wait()

        def hist_body(i):
            sv = st_v[pl.ds(i * L, L)]
            plsc.addupdate_scatter(blk_v, [sv, lanes], ones)

        plsc.parallel_loop(0, NVEC, 1, unroll=UNROLL)(hist_body)

        d_ph.wait()

        def mom_body(i, cr):
            a1, a2 = cr
            pv = ph_v[pl.ds(i * L, L)]
            return (a1 + pv, a2 + pv * pv)

        a1, a2 = plsc.parallel_loop(
            0, NVEC, 1, unroll=UNROLL, carry=(zero, zero))(mom_body)
        blk_v[NSTATES] = a1
        blk_v[NSTATES + 1] = a2

        pltpu.sync_copy(blk_v, out_hbm.at[:, w])

    return _sc_partials


def _poly(t, coeffs):
    acc = jnp.float32(coeffs[-1])
    for c in coeffs[-2::-1]:
        acc = acc * t + jnp.float32(c)
    return acc


_I0_SMALL = (1.0, 3.5156229, 3.0899424, 1.2067492, 0.2659732,
             0.0360768, 0.0045813)
_I0_LARGE = (0.39894228, 0.01328592, 0.00225319, -0.00157565, 0.00916281,
             -0.02057706, 0.02635537, -0.01647633, 0.00392377)
_I1_SMALL = (0.5, 0.87890594, 0.51498869, 0.15084934, 0.02658733,
             0.00301532, 0.00032411)
_I1_LARGE = (0.39894228, -0.03988024, -0.00362018, 0.00163801, -0.01031555,
             0.02282967, -0.02895312, 0.01787654, -0.00420059)


def _i0e(x):
    xs = jnp.minimum(x, 3.75)
    xl = jnp.maximum(x, 3.75)
    small = _poly((xs / 3.75) ** 2, _I0_SMALL) * jnp.exp(-xs)
    large = _poly(3.75 / xl, _I0_LARGE) / jnp.sqrt(xl)
    return jnp.where(x < 3.75, small, large)


def _i1e(x):
    xs = jnp.minimum(x, 3.75)
    xl = jnp.maximum(x, 3.75)
    small = xs * _poly((xs / 3.75) ** 2, _I1_SMALL) * jnp.exp(-xs)
    large = _poly(3.75 / xl, _I1_LARGE) / jnp.sqrt(xl)
    return jnp.where(x < 3.75, small, large)


def _log_i0e(x):
    return jnp.log(_i0e(x))


LOG_2PI = 1.8378770664093453
N_POW = float(NTOT) ** (-0.2)


def _tc_tail(red_ref, emb_ref, lng_ref, lnb_ref, w1_ref, b1_ref,
             wk_ref, bk_ref, o_mi, o_hz, o_hphi, o_hcond, o_bdc):
    red = red_ref[...]
    sums = jnp.sum(red, axis=(1, 2))
    counts = sums[:NSTATES]
    s1 = sums[NSTATES]
    s2 = sums[NSTATES + 1]
    n = jnp.float32(NTOT)

    probs = counts / n + 1e-10
    h_z = -jnp.sum(probs * jnp.log(probs))

    var_p = (s2 - s1 * s1 / n) / (n - 1.0)
    std_p = jnp.sqrt(jnp.maximum(var_p, 0.0))
    bw = 1.06 * std_p * N_POW
    kap_kde = jnp.minimum(1.0 / (bw * bw + 1e-6), 100.0)
    h_phi = LOG_2PI + _log_i0e(kap_kde) + kap_kde

    h = emb_ref[...]
    mean = jnp.mean(h, axis=1, keepdims=True)
    var = jnp.mean((h - mean) ** 2, axis=1, keepdims=True)
    h = (h - mean) / jnp.sqrt(var + 1e-5) * lng_ref[...] + lnb_ref[...]
    h = jnp.maximum(h, 0.0)
    h = lax.dot_general(h, w1_ref[...], (((1,), (1,)), ((), ())),
                        preferred_element_type=jnp.float32) + b1_ref[...]
    h = jnp.maximum(h, 0.0)
    kp = lax.dot_general(h, wk_ref[...], (((1,), (1,)), ((), ())),
                         preferred_element_type=jnp.float32)[:, 0] + bk_ref[...]
    kappa = jnp.maximum(kp, 0.0) + jnp.log1p(jnp.exp(-jnp.abs(kp))) + 0.1

    i0e_k = _i0e(kappa)
    ratio = _i1e(kappa) / i0e_k
    h_vm = LOG_2PI + jnp.log(i0e_k) + kappa - kappa * ratio
    h_cond = jnp.sum((counts / n) * h_vm)

    mi = h_phi - h_cond
    bdc = jnp.clip(2.0 * mi / (h_z + h_phi + 1e-12), 0.0, 1.0)

    o_mi[...] = mi
    o_hz[...] = h_z
    o_hphi[...] = h_phi
    o_hcond[...] = h_cond
    o_bdc[...] = bdc


def kernel(states, phases, emb, ln_g, ln_b, W1, b1, Wmu, bmu, Wk, bk):
    red = _build_sc_partials()(states, phases)
    scalar = jax.ShapeDtypeStruct((), jnp.float32)
    out = pl.pallas_call(
        _tc_tail,
        out_shape=[scalar] * 5,
        out_specs=[pl.BlockSpec(memory_space=pltpu.SMEM)] * 5,
    )(red, emb, ln_g, ln_b, W1, b1, Wk, bk)
    return tuple(out)


# speedup vs baseline: 1.0639x; 1.0024x over previous
import functools

import jax
import jax.numpy as jnp
from jax import lax
from jax.experimental import pallas as pl
from jax.experimental.pallas import tpu as pltpu
from jax.experimental.pallas import tpu_sc as plsc

NSTATES = 32
HID = 64
NTOT = 1048576
NC = 2
NS = 16
L = 16
NW = NC * NS
PER_W = NTOT // NW
NVEC = PER_W // L
ROWS = NSTATES + 2
UNROLL = 8

NCH = 8
CHN = PER_W // NCH
CVEC = CHN // L


@functools.cache
def _build_sc_partials():
    mesh = plsc.VectorSubcoreMesh(
        core_axis_name="c", subcore_axis_name="s",
        num_cores=NC, num_subcores=NS)

    @functools.partial(
        pl.kernel,
        out_type=jax.ShapeDtypeStruct((NW, ROWS, L), jnp.float32),
        mesh=mesh,
        scratch_types=[
            pltpu.VMEM((PER_W,), jnp.int32),
            pltpu.VMEM((PER_W,), jnp.float32),
            pltpu.VMEM((ROWS, L), jnp.float32),
            pltpu.SemaphoreType.DMA,
            pltpu.SemaphoreType.DMA,
        ],
        compiler_params=pltpu.CompilerParams(needs_layout_passes=False),
    )
    def _sc_partials(states_hbm, phases_hbm, out_hbm, st_v, ph_v, blk_v,
                     sem0, sem1):
        c = lax.axis_index("c")
        s = lax.axis_index("s")
        w = s * NC + c
        base = w * PER_W
        d_st = pltpu.async_copy(
            states_hbm.at[pl.ds(base, PER_W)], st_v, sem0)
        d_ph = pltpu.async_copy(
            phases_hbm.at[pl.ds(base, PER_W)], ph_v, sem1)

        zero = jnp.zeros((L,), jnp.float32)
        for r in range(ROWS):
            blk_v[r] = zero
        lanes = lax.broadcasted_iota(jnp.int32, (L,), 0)
        ones = jnp.ones((L,), jnp.float32)

        d_st.wait()

        def hist_body(i):
            sv = st_v[pl.ds(i * L, L)]
            plsc.addupdate_scatter(blk_v, [sv, lanes], ones)

        plsc.parallel_loop(0, NVEC, 1, unroll=UNROLL)(hist_body)

        d_ph.wait()

        def mom_body(i, cr):
            a1, a2 = cr
            pv = ph_v[pl.ds(i * L, L)]
            return (a1 + pv, a2 + pv * pv)

        a1, a2 = plsc.parallel_loop(
            0, NVEC, 1, unroll=UNROLL, carry=(zero, zero))(mom_body)
        blk_v[NSTATES] = a1
        blk_v[NSTATES + 1] = a2

        pltpu.sync_copy(blk_v, out_hbm.at[w])

    return _sc_partials


def _poly(t, coeffs):
    acc = jnp.float32(coeffs[-1])
    for c in coeffs[-2::-1]:
        acc = acc * t + jnp.float32(c)
    return acc


_I0_SMALL = (1.0, 3.5156229, 3.0899424, 1.2067492, 0.2659732,
             0.0360768, 0.0045813)
_I0_LARGE = (0.39894228, 0.01328592, 0.00225319, -0.00157565, 0.00916281,
             -0.02057706, 0.02635537, -0.01647633, 0.00392377)
_I1_SMALL = (0.5, 0.87890594, 0.51498869, 0.15084934, 0.02658733,
             0.00301532, 0.00032411)
_I1_LARGE = (0.39894228, -0.03988024, -0.00362018, 0.00163801, -0.01031555,
             0.02282967, -0.02895312, 0.01787654, -0.00420059)


def _i0e(x):
    xs = jnp.minimum(x, 3.75)
    xl = jnp.maximum(x, 3.75)
    small = _poly((xs / 3.75) ** 2, _I0_SMALL) * jnp.exp(-xs)
    large = _poly(3.75 / xl, _I0_LARGE) / jnp.sqrt(xl)
    return jnp.where(x < 3.75, small, large)


def _i1e(x):
    xs = jnp.minimum(x, 3.75)
    xl = jnp.maximum(x, 3.75)
    small = xs * _poly((xs / 3.75) ** 2, _I1_SMALL) * jnp.exp(-xs)
    large = _poly(3.75 / xl, _I1_LARGE) / jnp.sqrt(xl)
    return jnp.where(x < 3.75, small, large)


def _log_i0e(x):
    return jnp.log(_i0e(x))


LOG_2PI = 1.8378770664093453
N_POW = float(NTOT) ** (-0.2)


def _tc_tail(red_ref, emb_ref, lng_ref, lnb_ref, w1_ref, b1_ref,
             wk_ref, bk_ref, o_mi, o_hz, o_hphi, o_hcond, o_bdc):
    red = red_ref[...]
    sums = jnp.sum(red, axis=(0, 2))
    counts = sums[:NSTATES]
    s1 = sums[NSTATES]
    s2 = sums[NSTATES + 1]
    n = jnp.float32(NTOT)

    probs = counts / n + 1e-10
    h_z = -jnp.sum(probs * jnp.log(probs))

    var_p = (s2 - s1 * s1 / n) / (n - 1.0)
    std_p = jnp.sqrt(jnp.maximum(var_p, 0.0))
    bw = 1.06 * std_p * N_POW
    kap_kde = jnp.minimum(1.0 / (bw * bw + 1e-6), 100.0)
    h_phi = LOG_2PI + _log_i0e(kap_kde) + kap_kde

    h = emb_ref[...]
    mean = jnp.mean(h, axis=1, keepdims=True)
    var = jnp.mean((h - mean) ** 2, axis=1, keepdims=True)
    h = (h - mean) / jnp.sqrt(var + 1e-5) * lng_ref[...] + lnb_ref[...]
    h = jnp.maximum(h, 0.0)
    h = lax.dot_general(h, w1_ref[...], (((1,), (1,)), ((), ())),
                        preferred_element_type=jnp.float32) + b1_ref[...]
    h = jnp.maximum(h, 0.0)
    kp = lax.dot_general(h, wk_ref[...], (((1,), (1,)), ((), ())),
                         preferred_element_type=jnp.float32)[:, 0] + bk_ref[...]
    kappa = jnp.maximum(kp, 0.0) + jnp.log1p(jnp.exp(-jnp.abs(kp))) + 0.1

    i0e_k = _i0e(kappa)
    ratio = _i1e(kappa) / i0e_k
    h_vm = LOG_2PI + jnp.log(i0e_k) + kappa - kappa * ratio
    h_cond = jnp.sum((counts / n) * h_vm)

    mi = h_phi - h_cond
    bdc = jnp.clip(2.0 * mi / (h_z + h_phi + 1e-12), 0.0, 1.0)

    o_mi[...] = mi
    o_hz[...] = h_z
    o_hphi[...] = h_phi
    o_hcond[...] = h_cond
    o_bdc[...] = bdc


def kernel(states, phases, emb, ln_g, ln_b, W1, b1, Wmu, bmu, Wk, bk):
    red = _build_sc_partials()(states, phases)
    scalar = jax.ShapeDtypeStruct((), jnp.float32)
    out = pl.pallas_call(
        _tc_tail,
        out_shape=[scalar] * 5,
        out_specs=[pl.BlockSpec(memory_space=pltpu.SMEM)] * 5,
    )(red, emb, ln_g, ln_b, W1, b1, Wk, bk)
    return tuple(out)
